# baseline (speedup 1.0000x reference)
import functools

import jax
import jax.numpy as jnp
from jax import lax
from jax.experimental import pallas as pl
from jax.experimental.pallas import tpu as pltpu
from jax.experimental.pallas import tpu_sc as plsc

D = 128
CHUNK = 128
NC = 2
NS = 16
NW = NC * NS
NBUF = 5
LA = 3


@functools.lru_cache(maxsize=None)
def _build(total):
  assert total % (NW * CHUNK) == 0
  b_per_w = total // NW
  n_chunks = b_per_w // CHUNK
  assert n_chunks % NBUF == 0 and n_chunks > NBUF
  mesh = plsc.VectorSubcoreMesh(core_axis_name="c", subcore_axis_name="s")

  @functools.partial(
      pl.kernel,
      out_type=jax.ShapeDtypeStruct((total, D), jnp.float32),
      mesh=mesh,
      scratch_types=[
          pltpu.VMEM((b_per_w,), jnp.int32),
          pltpu.VMEM((NBUF, CHUNK, D), jnp.float32),
      ] + [pltpu.SemaphoreType.DMA] * (2 * NBUF),
  )
  def gather_kernel(table_hbm, ids_hbm, out_hbm, idx_v, rows_v, *sems):
    gsem = sems[:NBUF]
    ssem = sems[NBUF:]
    wid = lax.axis_index("s") * NC + lax.axis_index("c")
    base = wid * b_per_w

    pltpu.sync_copy(ids_hbm.at[pl.ds(base, b_per_w)], idx_v)

    def start_gather(c, slot):
      pltpu.async_copy(
          table_hbm.at[idx_v.at[pl.ds(c * CHUNK, CHUNK)]],
          rows_v.at[slot], gsem[slot])

    def wait_gather(slot):
      pltpu.make_async_copy(
          table_hbm.at[pl.ds(0, CHUNK)], rows_v.at[slot], gsem[slot]).wait()

    def wait_store(slot):
      pltpu.make_async_copy(
          rows_v.at[slot], out_hbm.at[pl.ds(0, CHUNK)], ssem[slot]).wait()

    for c in range(LA):
      start_gather(c, c % NBUF)

    def outer(g, carry):
      c0 = g * NBUF
      for b in range(NBUF):
        c = c0 + b
        sg = (b + LA) % NBUF

        @pl.when(c + LA < n_chunks)
        def _():
          @pl.when(c >= NBUF - LA)
          def _():
            wait_store(sg)
          start_gather(c + LA, sg)

        wait_gather(b)
        pltpu.async_copy(
            rows_v.at[b], out_hbm.at[pl.ds(base + c * CHUNK, CHUNK)], ssem[b])
      return carry

    lax.fori_loop(0, n_chunks // NBUF, outer, 0)

    for b in range(NBUF):
      wait_store(b)

  return gather_kernel


def kernel(ids, embedding_matrix):
  b, h = ids.shape
  vocab, d = embedding_matrix.shape
  assert d == D
  flat = ids.reshape(-1).astype(jnp.int32)
  out = _build(b * h)(embedding_matrix, flat)
  return out.reshape(b, h, d)


# speedup vs baseline: 1.8485x; 1.8485x over previous
import functools

import jax
import jax.numpy as jnp
from jax import lax
from jax.experimental import pallas as pl
from jax.experimental.pallas import tpu as pltpu
from jax.experimental.pallas import tpu_sc as plsc

D = 128
H = 50
R = 2
NC = 2
NS = 16
NW = NC * NS
NBUF = 4
LA = 2


@functools.lru_cache(maxsize=None)
def _build(batch):
  assert batch % (NW * R) == 0
  rows_per_w = batch // NW
  n_chunks = rows_per_w // R
  assert n_chunks % NBUF == 0 and n_chunks > NBUF
  mesh = plsc.VectorSubcoreMesh(core_axis_name="c", subcore_axis_name="s")

  @functools.partial(
      pl.kernel,
      out_type=jax.ShapeDtypeStruct((batch, H, D), jnp.float32),
      mesh=mesh,
      scratch_types=[
          pltpu.VMEM((rows_per_w, H), jnp.int32),
          pltpu.VMEM((NBUF, R, H, D), jnp.float32),
      ] + [pltpu.SemaphoreType.DMA] * (2 * NBUF),
  )
  def gather_kernel(table_hbm, ids_hbm, out_hbm, idx_v, rows_v, *sems):
    gsem = sems[:NBUF]
    ssem = sems[NBUF:]
    wid = lax.axis_index("s") * NC + lax.axis_index("c")
    base = wid * rows_per_w

    pltpu.sync_copy(ids_hbm.at[pl.ds(base, rows_per_w)], idx_v)

    def start_gathers(c, slot):
      for j in range(R):
        pltpu.async_copy(
            table_hbm.at[idx_v.at[c * R + j]],
            rows_v.at[slot].at[j], gsem[slot])

    def wait_gathers(slot):
      pltpu.make_async_copy(
          out_hbm.at[pl.ds(0, R)],
          rows_v.at[slot], gsem[slot]).wait()

    def wait_store(slot):
      pltpu.make_async_copy(
          rows_v.at[slot], out_hbm.at[pl.ds(0, R)], ssem[slot]).wait()

    for c in range(LA):
      start_gathers(c, c % NBUF)

    def outer(g, carry):
      c0 = g * NBUF
      for b in range(NBUF):
        c = c0 + b
        sg = (b + LA) % NBUF

        @pl.when(c + LA < n_chunks)
        def _():
          @pl.when(c >= NBUF - LA)
          def _():
            wait_store(sg)
          start_gathers(c + LA, sg)

        wait_gathers(b)
        pltpu.async_copy(
            rows_v.at[b], out_hbm.at[pl.ds(base + c * R, R)], ssem[b])
      return carry

    lax.fori_loop(0, n_chunks // NBUF, outer, 0)

    for b in range(NBUF):
      wait_store(b)

  return gather_kernel


def kernel(ids, embedding_matrix):
  b, h = ids.shape
  vocab, d = embedding_matrix.shape
  assert h == H and d == D
  return _build(b)(embedding_matrix, ids.astype(jnp.int32))
